# Initial kernel scaffold; baseline (speedup 1.0000x reference)
import jax
import jax.numpy as jnp
from jax.experimental import pallas as pl


def kernel(inputs, token_table, pos_embedding):
    raise NotImplementedError("write your pallas kernel here")


# speedup vs baseline: 2.9561x; 2.9561x over previous
import functools

import jax
import jax.numpy as jnp
from jax import lax
from jax.experimental import pallas as pl
from jax.experimental.pallas import tpu as pltpu
from jax.experimental.pallas import tpu_sc as plsc

LANES = 16


@functools.lru_cache(maxsize=None)
def _make_sc_embed(NW, NC, CHUNKS, CROWS, D, NBUF):
    mesh = plsc.VectorSubcoreMesh(core_axis_name="c", subcore_axis_name="s")
    NVEC = D // LANES
    NGROUPS = CHUNKS // NBUF

    scratch = [
        pltpu.VMEM((CHUNKS, CROWS), jnp.int32),
        pltpu.VMEM((CROWS, D), jnp.float32),
    ]
    scratch += [pltpu.VMEM((CROWS, D), jnp.float32) for _ in range(NBUF)]
    scratch += [pltpu.SemaphoreType.DMA for _ in range(2 * NBUF)]

    @functools.partial(
        pl.kernel,
        mesh=mesh,
        out_type=jax.ShapeDtypeStruct((NW, CHUNKS, CROWS, D), jnp.float32),
        scratch_types=scratch,
    )
    def sc_embed(idx_hbm, table_hbm, pe_hbm, out_hbm, idx_v, pe_v, *rest):
        bufs = rest[:NBUF]
        gsems = rest[NBUF:2 * NBUF]
        wsems = rest[2 * NBUF:3 * NBUF]

        wid = lax.axis_index("s") * NC + lax.axis_index("c")

        pltpu.sync_copy(idx_hbm.at[wid], idx_v)
        pltpu.sync_copy(pe_hbm, pe_v)

        def gather_start(g, b):
            pltpu.async_copy(table_hbm.at[idx_v.at[g]], bufs[b], gsems[b])

        def gather_wait(g, b):
            pltpu.make_async_copy(
                table_hbm.at[idx_v.at[g]], bufs[b], gsems[b]).wait()

        def write_start(g, b):
            pltpu.async_copy(bufs[b], out_hbm.at[wid, g], wsems[b])

        def write_wait(g, b):
            pltpu.make_async_copy(
                bufs[b], out_hbm.at[wid, g], wsems[b]).wait()

        def add_pe(b):
            buf = bufs[b]

            def body(i, carry):
                for j in range(4):
                    r = i * 4 + j
                    for v in range(NVEC):
                        sl = pl.ds(v * LANES, LANES)
                        plsc.addupdate(buf.at[r, sl], pe_v[r, sl])
                return carry

            lax.fori_loop(0, CROWS // 4, body, 0, unroll=False)

        for b in range(NBUF):
            gather_start(b, b)

        def group(o, carry):
            base = o * NBUF
            for b in range(NBUF):
                gather_wait(base + b, b)
                add_pe(b)
                write_start(base + b, b)
            nxt = base + NBUF
            for b in range(NBUF):
                write_wait(base + b, b)
                gather_start(nxt + b, b)
            return carry

        lax.fori_loop(0, NGROUPS - 1, group, 0, unroll=False)

        base = (NGROUPS - 1) * NBUF
        for b in range(NBUF):
            gather_wait(base + b, b)
            add_pe(b)
            write_start(base + b, b)
        for b in range(NBUF):
            write_wait(base + b, b)

    return sc_embed


def kernel(inputs, token_table, pos_embedding):
    B, K, S = inputs.shape
    V, D = token_table.shape

    info = plsc.get_sparse_core_info()
    NW = info.num_cores * info.num_subcores
    NC = info.num_cores

    R = B * K * S
    per_w = R // NW
    CROWS = 2 * S
    CHUNKS = per_w // CROWS
    NBUF = 4

    idx = inputs.reshape(NW, CHUNKS, CROWS)
    pe_seq = pos_embedding[:S]
    pe2 = jnp.concatenate([pe_seq, pe_seq], axis=0)

    fn = _make_sc_embed(NW, NC, CHUNKS, CROWS, D, NBUF)
    out = fn(idx, token_table, pe2)
    return out.reshape(B, K, S, D)


# speedup vs baseline: 4.7558x; 1.6088x over previous
import functools

import jax
import jax.numpy as jnp
from jax import lax
from jax.experimental import pallas as pl
from jax.experimental.pallas import tpu as pltpu
from jax.experimental.pallas import tpu_sc as plsc

LANES = 16


@functools.lru_cache(maxsize=None)
def _make_sc_embed(B, K, S, NW, NC, D, NBUF):
    mesh = plsc.VectorSubcoreMesh(core_axis_name="c", subcore_axis_name="s")
    NVEC = D // LANES
    CHUNKS = (B * K) // NW
    NGROUPS = CHUNKS // NBUF

    scratch = [
        pltpu.VMEM((CHUNKS, S), jnp.int32),
        pltpu.VMEM((S, D), jnp.float32),
    ]
    scratch += [pltpu.VMEM((S, D), jnp.float32) for _ in range(NBUF)]
    scratch += [pltpu.SemaphoreType.DMA for _ in range(2 * NBUF)]

    @functools.partial(
        pl.kernel,
        mesh=mesh,
        out_type=jax.ShapeDtypeStruct((B, K, S, D), jnp.float32),
        scratch_types=scratch,
    )
    def sc_embed(idx_hbm, table_hbm, pe_hbm, out_hbm, idx_v, pe_v, *rest):
        bufs = rest[:NBUF]
        gsems = rest[NBUF:2 * NBUF]
        wsems = rest[2 * NBUF:3 * NBUF]

        wid = lax.axis_index("s") * NC + lax.axis_index("c")

        pltpu.sync_copy(idx_hbm.at[wid], idx_v)
        pltpu.sync_copy(pe_hbm, pe_v)

        def gather_start(g, b):
            pltpu.async_copy(table_hbm.at[idx_v.at[g]], bufs[b], gsems[b])

        def gather_wait(g, b):
            pltpu.make_async_copy(
                table_hbm.at[idx_v.at[g]], bufs[b], gsems[b]).wait()

        def out_window(g):
            p = wid * CHUNKS + g
            return out_hbm.at[lax.div(p, K), lax.rem(p, K)]

        def write_start(g, b):
            pltpu.async_copy(bufs[b], out_window(g), wsems[b])

        def write_wait(g, b):
            pltpu.make_async_copy(bufs[b], out_window(g), wsems[b]).wait()

        def add_pe(b):
            buf = bufs[b]

            def body(i, carry):
                for j in range(2):
                    r = i * 2 + j
                    for v in range(NVEC):
                        sl = pl.ds(v * LANES, LANES)
                        plsc.addupdate(buf.at[r, sl], pe_v[r, sl])
                return carry

            lax.fori_loop(0, S // 2, body, 0, unroll=False)

        for b in range(NBUF):
            gather_start(b, b)

        def group(o, carry):
            base = o * NBUF
            for b in range(NBUF):
                gather_wait(base + b, b)
                add_pe(b)
                write_start(base + b, b)
            nxt = base + NBUF
            for b in range(NBUF):
                write_wait(base + b, b)
                gather_start(nxt + b, b)
            return carry

        lax.fori_loop(0, NGROUPS - 1, group, 0, unroll=False)

        base = (NGROUPS - 1) * NBUF
        for b in range(NBUF):
            gather_wait(base + b, b)
            add_pe(b)
            write_start(base + b, b)
        for b in range(NBUF):
            write_wait(base + b, b)

    return sc_embed


def kernel(inputs, token_table, pos_embedding):
    B, K, S = inputs.shape
    V, D = token_table.shape

    info = plsc.get_sparse_core_info()
    NW = info.num_cores * info.num_subcores
    NC = info.num_cores

    CHUNKS = (B * K) // NW
    NBUF = 4

    idx = inputs.reshape(NW, CHUNKS, S)
    pe_seq = pos_embedding[:S]

    fn = _make_sc_embed(B, K, S, NW, NC, D, NBUF)
    return fn(idx, token_table, pe_seq)
